# Initial kernel scaffold; baseline (speedup 1.0000x reference)
import jax
import jax.numpy as jnp
from jax.experimental import pallas as pl


def kernel(poses):
    raise NotImplementedError("write your pallas kernel here")


# speedup vs baseline: 1.5272x; 1.5272x over previous
import numpy as np
import jax
import jax.numpy as jnp
from jax import lax
from jax.experimental import pallas as pl
from jax.experimental.pallas import tpu as pltpu

_FACE_INDS = np.array([[0, 1, 2, 3], [0, 4, 2, 6], [0, 1, 4, 5],
                       [1, 3, 5, 7], [2, 3, 6, 7], [4, 5, 6, 7]])
_NG = 11
_NP = _NG * _NG
_NF = 6
_NCT = 10
_NH = 21
_NK = 29
_ITA = 0.2
_HF = _NH * _NF


def _grid_np():
    g = []
    for u in np.linspace(0.0, 1.0, _NG):
        for v in np.linspace(0.0, 1.0, _NG):
            g.append([v * (1.0 - u), v * u, (1.0 - v) * (1.0 - u), u * (1.0 - v)])
    return np.array(g, dtype=np.float32)


_OFF_HREP = 0
_OFF_OBJF = 384
_OFF_OBJFS = 1920
_OFF_P1 = 2016
_OFF_P2 = 2019
_OFF_EDGE = 2022
_OFF_DIR = 2046
_S_ROWS = 2056


def _build_s():
    s = np.zeros((_S_ROWS, _NK * 3), dtype=np.float32)
    for c in range(3):
        for hf in range(_HF):
            h = hf // _NF
            s[_OFF_HREP + c * 128 + hf, h * 3 + c] = 1.0
        for j in range(4):
            for hf in range(_HF):
                f = hf % _NF
                k = 21 + _FACE_INDS[f, j]
                s[_OFF_OBJF + (c * 4 + j) * 128 + hf, k * 3 + c] = 1.0
            for f in range(_NF):
                k = 21 + _FACE_INDS[f, j]
                s[_OFF_OBJFS + (c * 4 + j) * 8 + f, k * 3 + c] = 1.0
        for k in range(4):
            s[_OFF_P1 + c, (21 + k) * 3 + c] = 0.25
            s[_OFF_P2 + c, (25 + k) * 3 + c] = 0.25
            s[_OFF_DIR + c, (21 + k) * 3 + c] = -0.25
            s[_OFF_DIR + c, (25 + k) * 3 + c] = 0.25
        for r in range(2):
            for j in range(4):
                i1 = 21 + 4 * r + j
                i2 = 21 + 4 * r + (j + 1) % 4
                s[_OFF_EDGE + (r * 4 + j) * 3 + c, i1 * 3 + c] = 1.0
                s[_OFF_EDGE + (r * 4 + j) * 3 + c, i2 * 3 + c] = -1.0
    return s


_S_NP = _build_s()
_G_NP = np.zeros((_NP, 8), dtype=np.float32)
_G_NP[:, :4] = _grid_np()


def _body(s_ref, g_ref, x_ref, num_ref, den_ref):
    B = x_ref.shape[1]
    xb = x_ref[...]
    y = jnp.dot(s_ref[...], xb, preferred_element_type=jnp.float32)
    g = g_ref[...]

    p_acc = None
    for c in range(3):
        hrep = y[_OFF_HREP + c * 128:_OFF_HREP + c * 128 + _HF][None]
        t = None
        for j in range(4):
            o = _OFF_OBJF + (c * 4 + j) * 128
            objf = y[o:o + _HF][None]
            gj = g[:, j][:, None, None]
            term = gj * objf
            t = term if t is None else t + term
        dlt = t - hrep
        p_acc = dlt * dlt if p_acc is None else p_acc + dlt * dlt

    mn = jnp.min(p_acc, axis=0)
    ii = lax.broadcasted_iota(jnp.int32, p_acc.shape, 0)
    am = jnp.min(jnp.where(p_acc <= mn[None], ii, _NP), axis=0)
    d = jnp.sqrt(mn + 1e-6)

    fps = []
    for c in range(3):
        t = None
        for j in range(4):
            o = _OFF_OBJFS + (c * 4 + j) * 8
            objfs = y[o:o + _NF][None]
            term = g[:, j][:, None, None] * objfs
            t = term if t is None else t + term
        fps.append(t)

    p1c = [y[_OFF_P1 + c:_OFF_P1 + c + 1] for c in range(3)]
    dirc = [y[_OFF_DIR + c:_OFF_DIR + c + 1] for c in range(3)]

    lens = []
    for r in range(2):
        acc = None
        for j in range(4):
            e2 = None
            for c in range(3):
                row = y[_OFF_EDGE + (r * 4 + j) * 3 + c:
                        _OFF_EDGE + (r * 4 + j) * 3 + c + 1]
                e2 = row * row if e2 is None else e2 + row * row
            en = jnp.sqrt(e2)
            acc = en if acc is None else acc + en
        lens.append(acc * 0.25)
    thr = (lens[0] + lens[1]) * 0.5 * _ITA

    dnorm = jnp.sqrt(dirc[0] ** 2 + dirc[1] ** 2 + dirc[2] ** 2)
    idn = 1.0 / (dnorm + 1e-5)

    sub126 = lax.broadcasted_iota(jnp.int32, (_HF, B), 0)
    pio = lax.broadcasted_iota(jnp.int32, (_NP, _NF, B), 0)
    fio = lax.broadcasted_iota(jnp.int32, (_NP, _NF, B), 1)

    wn = [jnp.zeros((1, B), jnp.float32) for _ in range(3)]
    msum = jnp.zeros((1, B), jnp.float32)
    dcur = d
    for _ in range(_NCT):
        m = jnp.min(dcur, axis=0, keepdims=True)
        idx = jnp.min(jnp.where(dcur <= m, sub126, _HF), axis=0, keepdims=True)
        oh = sub126 == idx
        dcur = jnp.where(oh, jnp.float32(np.inf), dcur)
        pstar = jnp.sum(jnp.where(oh, am, 0), axis=0, keepdims=True)
        fstar = idx % _NF
        oh2 = (pio == pstar.reshape(1, 1, B)) & (fio == fstar.reshape(1, 1, B))
        coord = [jnp.sum(jnp.where(oh2, fps[c], 0.0), axis=(0, 1))[None]
                 for c in range(3)]
        mi = (m < thr).astype(jnp.float32)
        inner = None
        for c in range(3):
            v = dirc[c] * (coord[c] - p1c[c])
            inner = v if inner is None else inner + v
        tpar = inner * idn
        nn2 = None
        nv = []
        for c in range(3):
            nvc = coord[c] - (p1c[c] + dirc[c] * idn * tpar)
            nv.append(nvc)
            nn2 = nvc * nvc if nn2 is None else nn2 + nvc * nvc
        inn = 1.0 / (jnp.sqrt(nn2) + 1e-5)
        for c in range(3):
            wn[c] = wn[c] + mi * nv[c] * inn
        msum = msum + mi

    numb = wn[0] ** 2 + wn[1] ** 2 + wn[2] ** 2
    denb = msum * msum

    @pl.when(pl.program_id(0) == 0)
    def _():
        num_ref[0, 0] = 0.0
        den_ref[0, 0] = 0.0

    num_ref[0, 0] += jnp.sum(numb)
    den_ref[0, 0] += jnp.sum(denb)


def kernel(poses):
    bs = poses.shape[0]
    blk = 128
    x = poses.astype(jnp.float32).transpose(1, 2, 0).reshape(_NK * 3, bs)
    num, den = pl.pallas_call(
        _body,
        grid=(bs // blk,),
        in_specs=[
            pl.BlockSpec((_S_ROWS, _NK * 3), lambda i: (0, 0)),
            pl.BlockSpec((_NP, 8), lambda i: (0, 0)),
            pl.BlockSpec((_NK * 3, blk), lambda i: (0, i)),
        ],
        out_specs=[
            pl.BlockSpec(memory_space=pltpu.SMEM),
            pl.BlockSpec(memory_space=pltpu.SMEM),
        ],
        out_shape=[
            jax.ShapeDtypeStruct((1, 1), jnp.float32),
            jax.ShapeDtypeStruct((1, 1), jnp.float32),
        ],
    )(jnp.asarray(_S_NP), jnp.asarray(_G_NP), x)
    return num[0, 0] / (den[0, 0] + 1.0)
